# baseline (speedup 1.0000x reference)
import functools

import jax
import jax.numpy as jnp
from jax import lax
from jax.experimental import pallas as pl
from jax.experimental.pallas import tpu as pltpu
from jax.experimental.pallas import tpu_sc as plsc

NC, NS, LANES = 2, 16, 16
NW = NC * NS

EMB = 64
ROW = 128
SEQ = 200
HIDDEN = 256
NUM_LABELS = 10


def _gather_copies(eh, idxs_v, e, rows_buf, sem):
    return (
        pltpu.make_async_copy(eh.at[idxs_v.at[pl.ds(e * SEQ, 128)]],
                              rows_buf.at[pl.ds(0, 128)], sem),
        pltpu.make_async_copy(eh.at[idxs_v.at[pl.ds(e * SEQ + 128, SEQ - 128)]],
                              rows_buf.at[pl.ds(128, SEQ - 128)], sem),
    )


def _reduce_rows(rows_buf, out_v, e):
    nacc = EMB // LANES
    unroll = 8

    def red(t, accs):
        for u in range(unroll):
            s = t * unroll + u
            accs = tuple(a + rows_buf[s, pl.ds(LANES * j, LANES)]
                         for j, a in enumerate(accs))
        return accs

    accs = lax.fori_loop(
        0, SEQ // unroll, red,
        tuple(jnp.zeros((LANES,), jnp.float32) for _ in range(nacc)))
    for j in range(nacc):
        out_v[pl.ds(e * EMB + LANES * j, LANES)] = accs[j]


def _pool_body(x0_hbm, x2_hbm, x3_hbm, ew_hbm, eb_hbm, et_hbm,
               o0_hbm, o1_hbm, o2_hbm,
               idxs_v, rows_a, rows_b, out_v, sem_i, sem_a, sem_b):
    batch = o0_hbm.shape[0] // EMB
    bpw = batch // NW
    wid = lax.axis_index("s") * NC + lax.axis_index("c")
    base = wid * bpw
    bufs = (rows_a, rows_b)
    sems = (sem_a, sem_b)
    for xh, eh, oh in ((x0_hbm, ew_hbm, o0_hbm),
                       (x2_hbm, eb_hbm, o1_hbm),
                       (x3_hbm, et_hbm, o2_hbm)):
        pltpu.async_copy(xh.at[pl.ds(base * SEQ, bpw * SEQ)],
                         idxs_v, sem_i).wait()
        for c in _gather_copies(eh, idxs_v, 0, bufs[0], sems[0]):
            c.start()

        @pl.loop(0, bpw // 2)
        def _(i, eh=eh):
            for p in range(2):
                e = 2 * i + p
                for c in _gather_copies(eh, idxs_v, e, bufs[p], sems[p]):
                    c.wait()
                nxt = e + 1

                @pl.when(nxt < bpw)
                def _():
                    for c in _gather_copies(eh, idxs_v, nxt,
                                            bufs[1 - p], sems[1 - p]):
                        c.start()

                _reduce_rows(bufs[p], out_v, e)

        pltpu.sync_copy(out_v, oh.at[pl.ds(base * EMB, bpw * EMB)])


def _sc_pool(x0f, x2f, x3f, ew, eb, et, batch):
    bpw = batch // NW
    mesh = plsc.VectorSubcoreMesh(core_axis_name="c", subcore_axis_name="s")
    out = jax.ShapeDtypeStruct((batch * EMB,), jnp.float32)
    return pl.kernel(
        _pool_body,
        out_type=(out, out, out),
        mesh=mesh,
        compiler_params=pltpu.CompilerParams(use_tc_tiling_on_sc=True),
        scratch_types=[
            pltpu.VMEM((bpw * SEQ,), jnp.int32),
            pltpu.VMEM((SEQ, ROW), jnp.float32),
            pltpu.VMEM((SEQ, ROW), jnp.float32),
            pltpu.VMEM((bpw * EMB,), jnp.float32),
            pltpu.SemaphoreType.DMA,
            pltpu.SemaphoreType.DMA,
            pltpu.SemaphoreType.DMA,
        ],
    )(x0f, x2f, x3f, ew, eb, et)


TW = 512


def _detile_body(src_hbm, tail_hbm, out_hbm, slab_v, out_v, sem):
    v_total = out_hbm.shape[0]
    nfull = v_total // TW
    tail = v_total - nfull * TW
    wid = lax.axis_index("s") * NC + lax.axis_index("c")

    @pl.loop(wid, nfull, step=NW)
    def _(sl):
        v0 = sl * TW
        pltpu.async_copy(src_hbm.at[:, pl.ds(v0, TW)], slab_v, sem).wait()

        @pl.loop(0, TW)
        def _(v):
            col = jnp.full((LANES,), v, dtype=jnp.int32)
            for k in range(EMB // LANES):
                rows = lax.iota(jnp.int32, LANES) + (LANES * k)
                vals = plsc.load_gather(slab_v, [rows, col])
                out_v[v, pl.ds(LANES * k, LANES)] = vals

        pltpu.async_copy(out_v, out_hbm.at[pl.ds(v0, TW)], sem).wait()

    if tail:
        @pl.when(wid == 0)
        def _():
            pltpu.async_copy(tail_hbm, out_v.at[pl.ds(0, tail)], sem).wait()
            pltpu.async_copy(out_v.at[pl.ds(0, tail)],
                             out_hbm.at[pl.ds(nfull * TW, tail)], sem).wait()


def _sc_detile(emb):
    v_total = emb.shape[0]
    nfull = v_total // TW
    tail = v_total - nfull * TW
    tail_rows = jnp.pad(emb[nfull * TW:], ((0, 0), (0, ROW - EMB)))
    mesh = plsc.VectorSubcoreMesh(core_axis_name="c", subcore_axis_name="s")
    return pl.kernel(
        _detile_body,
        out_type=jax.ShapeDtypeStruct((v_total, ROW), jnp.float32),
        mesh=mesh,
        compiler_params=pltpu.CompilerParams(use_tc_tiling_on_sc=True,
                                             needs_layout_passes=False),
        scratch_types=[
            pltpu.VMEM((EMB, TW), jnp.float32),
            pltpu.VMEM((TW, ROW), jnp.float32),
            pltpu.SemaphoreType.DMA,
        ],
    )(emb.T, tail_rows)


def _mlp_body(p0_ref, p1_ref, p2_ref, w1_ref, b1_ref, w2_ref, b2_ref, o_ref):
    x = jnp.concatenate([p0_ref[...], p1_ref[...], p2_ref[...]], axis=1)
    h = jnp.dot(x, w1_ref[...], preferred_element_type=jnp.float32)
    h = h * (1.0 / SEQ) + b1_ref[...]
    h = jnp.maximum(h, 0.0)
    o_ref[...] = (jnp.dot(h, w2_ref[...], preferred_element_type=jnp.float32)
                  + b2_ref[...])


def _tc_mlp(p0, p1, p2, W1, b1, W2, b2):
    batch = p0.shape[0]
    bt = 512
    grid = (batch // bt,)
    return pl.pallas_call(
        _mlp_body,
        grid=grid,
        in_specs=[
            pl.BlockSpec((bt, EMB), lambda i: (i, 0)),
            pl.BlockSpec((bt, EMB), lambda i: (i, 0)),
            pl.BlockSpec((bt, EMB), lambda i: (i, 0)),
            pl.BlockSpec((3 * EMB, HIDDEN), lambda i: (0, 0)),
            pl.BlockSpec((1, HIDDEN), lambda i: (0, 0)),
            pl.BlockSpec((HIDDEN, NUM_LABELS), lambda i: (0, 0)),
            pl.BlockSpec((1, NUM_LABELS), lambda i: (0, 0)),
        ],
        out_specs=pl.BlockSpec((bt, NUM_LABELS), lambda i: (i, 0)),
        out_shape=jax.ShapeDtypeStruct((batch, NUM_LABELS), jnp.float32),
    )(p0, p1, p2, W1, b1, W2, b2)


def kernel(x0, x1, x2, x3, emb_word, emb_bi, emb_tri, W1, b1, W2, b2):
    del x1
    batch = x0.shape[0]
    ew = _sc_detile(emb_word)
    eb = _sc_detile(emb_bi)
    et = _sc_detile(emb_tri)
    x0f = x0.astype(jnp.int32).reshape(-1)
    x2f = x2.reshape(-1)
    x3f = x3.reshape(-1)
    f0, f1, f2 = _sc_pool(x0f, x2f, x3f, ew, eb, et, batch)
    p0 = f0.reshape(batch, EMB)
    p1 = f1.reshape(batch, EMB)
    p2 = f2.reshape(batch, EMB)
    return _tc_mlp(p0, p1, p2, W1, b1.reshape(1, HIDDEN),
                   W2, b2.reshape(1, NUM_LABELS))


# speedup vs baseline: 3.4241x; 3.4241x over previous
import functools

import jax
import jax.numpy as jnp
from jax import lax
from jax.experimental import pallas as pl
from jax.experimental.pallas import tpu as pltpu
from jax.experimental.pallas import tpu_sc as plsc

NC, NS, LANES = 2, 16, 16
NW = NC * NS

EMB = 64
SEQ = 200
HIDDEN = 256
NUM_LABELS = 10


def _gather_copies(eh, idxs_v, e, rows_buf, sem):
    return (
        pltpu.make_async_copy(eh.at[idxs_v.at[e, pl.ds(0, 128)]],
                              rows_buf.at[pl.ds(0, 128)], sem),
        pltpu.make_async_copy(eh.at[idxs_v.at[e, pl.ds(128, SEQ - 128)]],
                              rows_buf.at[pl.ds(128, SEQ - 128)], sem),
    )


def _reduce_rows(rows_buf, out_v, e):
    nacc = EMB // LANES
    unroll = 8

    def red(t, accs):
        for u in range(unroll):
            s = t * unroll + u
            accs = tuple(a + rows_buf[s, pl.ds(LANES * j, LANES)]
                         for j, a in enumerate(accs))
        return accs

    accs = lax.fori_loop(
        0, SEQ // unroll, red,
        tuple(jnp.zeros((LANES,), jnp.float32) for _ in range(nacc)))
    for j in range(nacc):
        out_v[e, pl.ds(LANES * j, LANES)] = accs[j]


def _pool_body(xh, eh, oh, idxs_v, rows_a, rows_b, out_v,
               sem_i, sem_a, sem_b):
    batch = xh.shape[0]
    bpw = batch // NW
    wid = lax.axis_index("s") * NC + lax.axis_index("c")
    base = wid * bpw
    bufs = (rows_a, rows_b)
    sems = (sem_a, sem_b)
    pltpu.async_copy(xh.at[pl.ds(base, bpw)], idxs_v, sem_i).wait()
    for c in _gather_copies(eh, idxs_v, 0, bufs[0], sems[0]):
        c.start()

    @pl.loop(0, bpw // 2)
    def _(i):
        for p in range(2):
            e = 2 * i + p
            for c in _gather_copies(eh, idxs_v, e, bufs[p], sems[p]):
                c.wait()
            nxt = e + 1

            @pl.when(nxt < bpw)
            def _():
                for c in _gather_copies(eh, idxs_v, nxt,
                                        bufs[1 - p], sems[1 - p]):
                    c.start()

            _reduce_rows(bufs[p], out_v, e)

    pltpu.sync_copy(out_v, oh.at[pl.ds(base, bpw)])


def _sc_pool_one(x, emb):
    batch = x.shape[0]
    bpw = batch // NW
    mesh = plsc.VectorSubcoreMesh(core_axis_name="c", subcore_axis_name="s")
    return pl.kernel(
        _pool_body,
        out_type=jax.ShapeDtypeStruct((batch, EMB), jnp.float32),
        mesh=mesh,
        compiler_params=pltpu.CompilerParams(use_tc_tiling_on_sc=False),
        scratch_types=[
            pltpu.VMEM((bpw, SEQ), jnp.int32),
            pltpu.VMEM((SEQ, EMB), jnp.float32),
            pltpu.VMEM((SEQ, EMB), jnp.float32),
            pltpu.VMEM((bpw, EMB), jnp.float32),
            pltpu.SemaphoreType.DMA,
            pltpu.SemaphoreType.DMA,
            pltpu.SemaphoreType.DMA,
        ],
    )(x, emb)


def _mlp_body(p0_ref, p1_ref, p2_ref, w1_ref, b1_ref, w2_ref, b2_ref, o_ref):
    x = jnp.concatenate([p0_ref[...], p1_ref[...], p2_ref[...]], axis=1)
    h = jnp.dot(x, w1_ref[...], preferred_element_type=jnp.float32)
    h = h * (1.0 / SEQ) + b1_ref[...]
    h = jnp.maximum(h, 0.0)
    o_ref[...] = (jnp.dot(h, w2_ref[...], preferred_element_type=jnp.float32)
                  + b2_ref[...])


def _tc_mlp(p0, p1, p2, W1, b1, W2, b2):
    batch = p0.shape[0]
    bt = 512
    grid = (batch // bt,)
    return pl.pallas_call(
        _mlp_body,
        grid=grid,
        in_specs=[
            pl.BlockSpec((bt, EMB), lambda i: (i, 0)),
            pl.BlockSpec((bt, EMB), lambda i: (i, 0)),
            pl.BlockSpec((bt, EMB), lambda i: (i, 0)),
            pl.BlockSpec((3 * EMB, HIDDEN), lambda i: (0, 0)),
            pl.BlockSpec((1, HIDDEN), lambda i: (0, 0)),
            pl.BlockSpec((HIDDEN, NUM_LABELS), lambda i: (0, 0)),
            pl.BlockSpec((1, NUM_LABELS), lambda i: (0, 0)),
        ],
        out_specs=pl.BlockSpec((bt, NUM_LABELS), lambda i: (i, 0)),
        out_shape=jax.ShapeDtypeStruct((batch, NUM_LABELS), jnp.float32),
    )(p0, p1, p2, W1, b1, W2, b2)


def kernel(x0, x1, x2, x3, emb_word, emb_bi, emb_tri, W1, b1, W2, b2):
    del x1
    p0 = _sc_pool_one(x0.astype(jnp.int32), emb_word)
    p1 = _sc_pool_one(x2, emb_bi)
    p2 = _sc_pool_one(x3, emb_tri)
    return _tc_mlp(p0, p1, p2, W1, b1.reshape(1, HIDDEN),
                   W2, b2.reshape(1, NUM_LABELS))
